# baseline (speedup 1.0000x reference)
import functools

import jax
import jax.numpy as jnp
from jax import lax
from jax.experimental import pallas as pl
from jax.experimental.pallas import tpu as pltpu
from jax.experimental.pallas import tpu_sc as plsc

N = 10000
E = 320000
R_REL = 3
NC, NS, L = 2, 16, 16
NW = NC * NS
EP = 327680
EPW = EP // NW
EPS = E // NS
CN = R_REL * N
CNP = 30720
CPT = CNP // NS
K = 80
CPW = EPW // K
NP = 10240
D128 = 128
MB = 16
BB = 2000
BB3 = 2048

_mesh = plsc.VectorSubcoreMesh(
    core_axis_name="c", subcore_axis_name="s", num_cores=NC, num_subcores=NS)

f32 = jnp.float32
i32 = jnp.int32


def _prep_body(src_hbm, dst_hbm, rel_hbm,
               gidx_hbm, se_hbm,
               cnt_v, bdst_v, brel_v, bsrc_v, mrg_v, inv_v, oge_v, ose_v,
               cnt_sh, inv_sh, sem):
    c = lax.axis_index("c")
    s = lax.axis_index("s")
    w = c * NS + s

    ones = jnp.ones((L,), f32)

    def zero_cnt(i, _):
        cnt_v[pl.ds(pl.multiple_of(i * L, L), L)] = jnp.zeros((L,), f32)
        return 0
    lax.fori_loop(0, CNP // L, zero_cnt, 0)

    def count_block(b, _):
        base = pl.multiple_of(s * EPS + b * BB, 8)
        pltpu.sync_copy(dst_hbm.at[pl.ds(base, BB)], bdst_v.at[pl.ds(0, BB)])
        pltpu.sync_copy(rel_hbm.at[pl.ds(base, BB)], brel_v.at[pl.ds(0, BB)])

        def count_chunk(i, _):
            o = pl.multiple_of(i * L, L)
            d = bdst_v[pl.ds(o, L)]
            r = brel_v[pl.ds(o, L)]
            cidx = r * N + d
            plsc.addupdate_scatter(cnt_v, [cidx], ones)
            return 0
        lax.fori_loop(0, BB // L, count_chunk, 0)
        return 0
    lax.fori_loop(0, EPS // BB, count_block, 0)

    pltpu.sync_copy(cnt_v, cnt_sh.at[s])
    plsc.subcore_barrier()

    for k2 in range(NS):
        pltpu.sync_copy(cnt_sh.at[k2, pl.ds(pl.multiple_of(s * CPT, 8), CPT)],
                        mrg_v.at[k2])

    def inv_chunk(j, _):
        o = pl.multiple_of(j * L, L)
        tot = mrg_v[0, pl.ds(o, L)]
        for k2 in range(1, NS):
            tot = tot + mrg_v[k2, pl.ds(o, L)]
        inv_v[pl.ds(o, L)] = 1.0 / jnp.maximum(tot, 1.0)
        return 0
    lax.fori_loop(0, CPT // L, inv_chunk, 0)

    pltpu.sync_copy(inv_v,
                    inv_sh.at[pl.ds(pl.multiple_of(s * CPT, 8), CPT)])
    plsc.subcore_barrier()

    inv_full_view = cnt_v
    pltpu.sync_copy(inv_sh, inv_full_view)

    lane = lax.iota(i32, L)

    def out_block(b, _):
        base = pl.multiple_of(w * EPW + b * BB3, 8)
        pltpu.sync_copy(src_hbm.at[pl.ds(base, BB3)], bsrc_v)
        pltpu.sync_copy(dst_hbm.at[pl.ds(base, BB3)], bdst_v)
        pltpu.sync_copy(rel_hbm.at[pl.ds(base, BB3)], brel_v)

        def out_chunk(i, _):
            o = pl.multiple_of(i * L, L)
            sv = bsrc_v[pl.ds(o, L)]
            d = bdst_v[pl.ds(o, L)]
            r = brel_v[pl.ds(o, L)]
            rn = r * N
            se = plsc.load_gather(inv_full_view, [rn + d])
            real = (base + o + lane) < E
            ose_v[pl.ds(o, L)] = jnp.where(real, se, 0.0)
            oge_v[pl.ds(o, L)] = rn + sv
            return 0
        lax.fori_loop(0, BB3 // L, out_chunk, 0)

        pltpu.sync_copy(ose_v, se_hbm.at[pl.ds(base, BB3)])
        pltpu.sync_copy(oge_v, gidx_hbm.at[pl.ds(base, BB3)])
        return 0
    lax.fori_loop(0, EPW // BB3, out_block, 0)


@functools.partial(
    pl.kernel,
    out_type=(jax.ShapeDtypeStruct((EP,), i32),
              jax.ShapeDtypeStruct((EP,), f32)),
    mesh=_mesh,
    compiler_params=pltpu.CompilerParams(needs_layout_passes=False),
    scratch_types=(
        pltpu.VMEM((CNP,), f32),
        pltpu.VMEM((BB3,), i32),
        pltpu.VMEM((BB3,), i32),
        pltpu.VMEM((BB3,), i32),
        pltpu.VMEM((NS, CPT), f32),
        pltpu.VMEM((CPT,), f32),
        pltpu.VMEM((BB3,), i32),
        pltpu.VMEM((BB3,), f32),
        pltpu.VMEM_SHARED((NS, CNP), f32),
        pltpu.VMEM_SHARED((CNP,), f32),
        pltpu.SemaphoreType.DMA,
    ),
)
def _prep(src_hbm, dst_hbm, rel_hbm, gidx_hbm, se_hbm, *rest):
    _prep_body(src_hbm, dst_hbm, rel_hbm, gidx_hbm, se_hbm, *rest)


def _agg_body(D, C0, yf_hbm, gidx_hbm, dst_hbm, se_hbm, part_hbm,
              gidx_v, dst_v, se_v, rows_v, rows2_v, rsc_v, rsc2_v, acc_sh,
              sg0, sg1, ss0, ss1):
    c = lax.axis_index("c")
    s = lax.axis_index("s")
    C1 = (EP // K) // NS - C0
    my_c = jnp.where(c == 0, C0, C1)
    my_base = c * NS * C0 + s * my_c

    for i in range(32):
        for d in range(D // L):
            rsc_v[i, pl.ds(d * L, L)] = jnp.zeros((L,), f32)

    def zero_slice(t, _):
        pltpu.sync_copy(
            rsc_v.at[pl.ds(0, 32)],
            acc_sh.at[pl.ds(pl.multiple_of(s * 640 + t * 32, 8), 32)])
        return 0
    lax.fori_loop(0, 20, zero_slice, 0)

    plsc.subcore_barrier()

    def scale(rv, out, sj):
        for g in range(K // L):
            sv = se_v[sj, pl.ds(g * L, L)]
            for i in range(L):
                sc = sv[i]
                row = g * L + i
                for d in range(D // L):
                    sl = pl.ds(d * L, L)
                    out[row, sl] = rv[row, sl] * sc

    def mblock(mb, _):
        ro = pl.multiple_of(my_base + mb * MB, 8)
        pltpu.sync_copy(gidx_hbm.at[pl.ds(ro, MB)], gidx_v)
        pltpu.sync_copy(dst_hbm.at[pl.ds(ro, MB)], dst_v)
        pltpu.sync_copy(se_hbm.at[pl.ds(ro, MB)], se_v)

        pltpu.async_copy(yf_hbm.at[gidx_v.at[0]], rows_v, sg0)

        def pair(j2, _):
            j0 = j2 * 2
            j1 = j0 + 1
            pltpu.async_copy(yf_hbm.at[gidx_v.at[j1]], rows2_v, sg1)
            pltpu.make_async_copy(yf_hbm.at[gidx_v.at[j0]], rows_v, sg0).wait()

            @pl.when(j2 > 0)
            def _():
                pltpu.make_async_copy(
                    rsc_v, acc_sh.at[dst_v.at[j0 - 2]], ss0).wait()
            scale(rows_v, rsc_v, j0)
            pltpu.async_copy(rsc_v, acc_sh.at[dst_v.at[j0]], ss0, add=True)

            @pl.when(j2 + 1 < MB // 2)
            def _():
                pltpu.async_copy(yf_hbm.at[gidx_v.at[j0 + 2]], rows_v, sg0)
            pltpu.make_async_copy(
                yf_hbm.at[gidx_v.at[j1]], rows2_v, sg1).wait()

            @pl.when(j2 > 0)
            def _():
                pltpu.make_async_copy(
                    rsc2_v, acc_sh.at[dst_v.at[j1 - 2]], ss1).wait()
            scale(rows2_v, rsc2_v, j1)
            pltpu.async_copy(rsc2_v, acc_sh.at[dst_v.at[j1]], ss1, add=True)
            return 0
        lax.fori_loop(0, MB // 2, pair, 0)
        pltpu.make_async_copy(rsc_v, acc_sh.at[dst_v.at[MB - 2]], ss0).wait()
        pltpu.make_async_copy(rsc2_v, acc_sh.at[dst_v.at[MB - 1]], ss1).wait()
        return 0
    lax.fori_loop(0, my_c // MB, mblock, 0)

    plsc.subcore_barrier()
    rbk = pl.multiple_of(s * 640, 8)
    pltpu.sync_copy(acc_sh.at[pl.ds(rbk, 640)],
                    part_hbm.at[c, pl.ds(rbk, 640)])


def _make_agg(D, C0=128):
    @functools.partial(
        pl.kernel,
        out_type=jax.ShapeDtypeStruct((NC, NP, D), f32),
        mesh=_mesh,
        compiler_params=pltpu.CompilerParams(needs_layout_passes=False),
        scratch_types=(
            pltpu.VMEM((MB, K), i32),
            pltpu.VMEM((MB, K), i32),
            pltpu.VMEM((MB, K), f32),
            pltpu.VMEM((K, D128), f32),
            pltpu.VMEM((K, D128), f32),
            pltpu.VMEM((K, D), f32),
            pltpu.VMEM((K, D), f32),
            pltpu.VMEM_SHARED((NP, D), f32),
            pltpu.SemaphoreType.DMA,
            pltpu.SemaphoreType.DMA,
            pltpu.SemaphoreType.DMA,
            pltpu.SemaphoreType.DMA,
        ),
    )
    def agg(yf, gidx, dst, se, part, *rest):
        _agg_body(D, C0, yf, gidx, dst, se, part, *rest)
    return agg


_agg_k = {D: _make_agg(128, C0=144) for D in (128, 64, 32)}


BN = 1000


def _dense_first_body(x_ref, w_ref, r_ref, b_ref, y_ref, xrb_ref):
    xb = x_ref[...]
    xrb_ref[...] = jnp.dot(xb, r_ref[...],
                           preferred_element_type=f32) + b_ref[...]
    for r in range(R_REL):
        y_ref[r] = jnp.dot(xb, w_ref[r], preferred_element_type=f32)


def _dense_next_body(dprev, xrb_ref, p_ref, w_ref, r_ref, b_ref,
                     y_ref, xrb2_ref):
    p = p_ref[0] + p_ref[1]
    if p.shape[1] < D128:
        p = jnp.concatenate(
            [p, jnp.zeros((BN, D128 - p.shape[1]), f32)], axis=1)
    h = jnp.maximum(xrb_ref[...] + p, 0.0)
    xrb2_ref[...] = jnp.dot(h, r_ref[...],
                            preferred_element_type=f32) + b_ref[...]
    for r in range(R_REL):
        y_ref[r] = jnp.dot(h, w_ref[r], preferred_element_type=f32)


def _dense_first(x, W, Rm, b):
    return pl.pallas_call(
        _dense_first_body,
        grid=(N // BN,),
        in_specs=[
            pl.BlockSpec((BN, D128), lambda i: (i, 0)),
            pl.BlockSpec((R_REL, D128, D128), lambda i: (0, 0, 0)),
            pl.BlockSpec((D128, D128), lambda i: (0, 0)),
            pl.BlockSpec((1, D128), lambda i: (0, 0)),
        ],
        out_specs=[
            pl.BlockSpec((R_REL, BN, D128), lambda i: (0, i, 0)),
            pl.BlockSpec((BN, D128), lambda i: (i, 0)),
        ],
        out_shape=[
            jax.ShapeDtypeStruct((R_REL, N, D128), f32),
            jax.ShapeDtypeStruct((N, D128), f32),
        ],
    )(x, W, Rm, b.reshape(1, D128))


def _dense_next(xrb, part, W, Rm, b, dprev):
    return pl.pallas_call(
        functools.partial(_dense_next_body, dprev),
        grid=(N // BN,),
        in_specs=[
            pl.BlockSpec((BN, D128), lambda i: (i, 0)),
            pl.BlockSpec((NC, BN, D128), lambda i: (0, i, 0)),
            pl.BlockSpec((R_REL, D128, D128), lambda i: (0, 0, 0)),
            pl.BlockSpec((D128, D128), lambda i: (0, 0)),
            pl.BlockSpec((1, D128), lambda i: (0, 0)),
        ],
        out_specs=[
            pl.BlockSpec((R_REL, BN, D128), lambda i: (0, i, 0)),
            pl.BlockSpec((BN, D128), lambda i: (i, 0)),
        ],
        out_shape=[
            jax.ShapeDtypeStruct((R_REL, N, D128), f32),
            jax.ShapeDtypeStruct((N, D128), f32),
        ],
    )(xrb, part, W, Rm, b.reshape(1, D128))


def _final_body(xrb_ref, p_ref, fw_ref, fb_ref, o_ref):
    p = p_ref[0] + p_ref[1]
    if p.shape[1] < D128:
        p = jnp.concatenate(
            [p, jnp.zeros((BN, D128 - p.shape[1]), f32)], axis=1)
    h = jnp.maximum(xrb_ref[...] + p, 0.0)
    z = jnp.sum(h * fw_ref[...].reshape(1, D128), axis=1, keepdims=True)
    o_ref[...] = jax.nn.sigmoid(z + fb_ref[...])


def _final(xrb, part, fcW, fcB):
    return pl.pallas_call(
        _final_body,
        grid=(N // BN,),
        in_specs=[
            pl.BlockSpec((BN, D128), lambda i: (i, 0)),
            pl.BlockSpec((NC, BN, D128), lambda i: (0, i, 0)),
            pl.BlockSpec((D128, 1), lambda i: (0, 0)),
            pl.BlockSpec((1, 1), lambda i: (0, 0)),
        ],
        out_specs=pl.BlockSpec((BN, 1), lambda i: (i, 0)),
        out_shape=jax.ShapeDtypeStruct((N, 1), f32),
    )(xrb, part, fcW, fcB.reshape(1, 1))


def _pad2(a, rows, cols):
    return jnp.pad(a, ((0, rows - a.shape[0]), (0, cols - a.shape[1])))


def kernel(x, edge_index, edge_type, W1, R1, B1, W2, R2, B2, W3, R3, B3,
           W4, R4, B4, fcW, fcB):
    src = jnp.pad(edge_index[0].astype(i32), (0, EP - E))
    dst = jnp.pad(edge_index[1].astype(i32), (0, EP - E))
    rel = jnp.pad(edge_type.astype(i32), (0, EP - E))

    gidx, se = _prep(src, dst, rel)
    gidx2 = gidx.reshape(EP // K, K)
    dst2 = dst.reshape(EP // K, K)
    se2 = se.reshape(EP // K, K)

    def padw(W, Rm, b):
        Wp = jnp.stack([_pad2(W[r], D128, D128) for r in range(R_REL)])
        return Wp, _pad2(Rm, D128, D128), jnp.pad(b, (0, D128 - b.shape[0]))

    layers = [padw(W1, R1, B1), padw(W2, R2, B2),
              padw(W3, R3, B3), padw(W4, R4, B4)]

    douts = [128, 64, 32, 32]
    y, xrb = _dense_first(x, *layers[0])
    part = _agg_k[douts[0]](y.reshape(R_REL * N, D128), gidx2, dst2, se2)
    for li, (Wp, Rp, bp) in enumerate(layers[1:], start=1):
        y, xrb = _dense_next(xrb, part, Wp, Rp, bp, douts[li - 1])
        part = _agg_k[douts[li]](y.reshape(R_REL * N, D128), gidx2, dst2, se2)
    fwp = jnp.pad(fcW, ((0, D128 - fcW.shape[0]), (0, 0)))
    return _final(xrb, part, fwp, fcB)


# speedup vs baseline: 1.0677x; 1.0677x over previous
import functools

import jax
import jax.numpy as jnp
from jax import lax
from jax.experimental import pallas as pl
from jax.experimental.pallas import tpu as pltpu
from jax.experimental.pallas import tpu_sc as plsc

N = 10000
E = 320000
R_REL = 3
NC, NS, L = 2, 16, 16
NW = NC * NS
EP = 327680
EPW = EP // NW
EPS = E // NS
CN = R_REL * N
CNP = 30720
CPT = CNP // NS
K = 80
CPW = EPW // K
NP = 10240
D128 = 128
MB = 16
BB = 2000
BB3 = 2048

_mesh = plsc.VectorSubcoreMesh(
    core_axis_name="c", subcore_axis_name="s", num_cores=NC, num_subcores=NS)

f32 = jnp.float32
i32 = jnp.int32


def _prep_body(src_hbm, dst_hbm, rel_hbm,
               gidx_hbm, se_hbm,
               cnt_v, bdst_v, brel_v, bsrc_v, mrg_v, inv_v, oge_v, ose_v,
               cnt_sh, inv_sh, sem):
    c = lax.axis_index("c")
    s = lax.axis_index("s")
    w = c * NS + s

    ones = jnp.ones((L,), f32)

    def zero_cnt(i, _):
        cnt_v[pl.ds(pl.multiple_of(i * L, L), L)] = jnp.zeros((L,), f32)
        return 0
    lax.fori_loop(0, CNP // L, zero_cnt, 0)

    def count_block(b, _):
        base = pl.multiple_of(s * EPS + b * BB, 8)
        pltpu.sync_copy(dst_hbm.at[pl.ds(base, BB)], bdst_v.at[pl.ds(0, BB)])
        pltpu.sync_copy(rel_hbm.at[pl.ds(base, BB)], brel_v.at[pl.ds(0, BB)])

        def count_chunk(i, _):
            o = pl.multiple_of(i * L, L)
            d = bdst_v[pl.ds(o, L)]
            r = brel_v[pl.ds(o, L)]
            cidx = r * N + d
            plsc.addupdate_scatter(cnt_v, [cidx], ones)
            return 0
        lax.fori_loop(0, BB // L, count_chunk, 0)
        return 0
    lax.fori_loop(0, EPS // BB, count_block, 0)

    pltpu.sync_copy(cnt_v, cnt_sh.at[s])
    plsc.subcore_barrier()

    for k2 in range(NS):
        pltpu.sync_copy(cnt_sh.at[k2, pl.ds(pl.multiple_of(s * CPT, 8), CPT)],
                        mrg_v.at[k2])

    def inv_chunk(j, _):
        o = pl.multiple_of(j * L, L)
        tot = mrg_v[0, pl.ds(o, L)]
        for k2 in range(1, NS):
            tot = tot + mrg_v[k2, pl.ds(o, L)]
        inv_v[pl.ds(o, L)] = 1.0 / jnp.maximum(tot, 1.0)
        return 0
    lax.fori_loop(0, CPT // L, inv_chunk, 0)

    pltpu.sync_copy(inv_v,
                    inv_sh.at[pl.ds(pl.multiple_of(s * CPT, 8), CPT)])
    plsc.subcore_barrier()

    inv_full_view = cnt_v
    pltpu.sync_copy(inv_sh, inv_full_view)

    lane = lax.iota(i32, L)

    def out_block(b, _):
        base = pl.multiple_of(w * EPW + b * BB3, 8)
        pltpu.sync_copy(src_hbm.at[pl.ds(base, BB3)], bsrc_v)
        pltpu.sync_copy(dst_hbm.at[pl.ds(base, BB3)], bdst_v)
        pltpu.sync_copy(rel_hbm.at[pl.ds(base, BB3)], brel_v)

        def out_chunk(i, _):
            o = pl.multiple_of(i * L, L)
            sv = bsrc_v[pl.ds(o, L)]
            d = bdst_v[pl.ds(o, L)]
            r = brel_v[pl.ds(o, L)]
            rn = r * N
            se = plsc.load_gather(inv_full_view, [rn + d])
            real = (base + o + lane) < E
            ose_v[pl.ds(o, L)] = jnp.where(real, se, 0.0)
            oge_v[pl.ds(o, L)] = rn + sv
            return 0
        lax.fori_loop(0, BB3 // L, out_chunk, 0)

        pltpu.sync_copy(ose_v, se_hbm.at[pl.ds(base, BB3)])
        pltpu.sync_copy(oge_v, gidx_hbm.at[pl.ds(base, BB3)])
        return 0
    lax.fori_loop(0, EPW // BB3, out_block, 0)


@functools.partial(
    pl.kernel,
    out_type=(jax.ShapeDtypeStruct((EP,), i32),
              jax.ShapeDtypeStruct((EP,), f32)),
    mesh=_mesh,
    compiler_params=pltpu.CompilerParams(needs_layout_passes=False),
    scratch_types=(
        pltpu.VMEM((CNP,), f32),
        pltpu.VMEM((BB3,), i32),
        pltpu.VMEM((BB3,), i32),
        pltpu.VMEM((BB3,), i32),
        pltpu.VMEM((NS, CPT), f32),
        pltpu.VMEM((CPT,), f32),
        pltpu.VMEM((BB3,), i32),
        pltpu.VMEM((BB3,), f32),
        pltpu.VMEM_SHARED((NS, CNP), f32),
        pltpu.VMEM_SHARED((CNP,), f32),
        pltpu.SemaphoreType.DMA,
    ),
)
def _prep(src_hbm, dst_hbm, rel_hbm, gidx_hbm, se_hbm, *rest):
    _prep_body(src_hbm, dst_hbm, rel_hbm, gidx_hbm, se_hbm, *rest)


def _agg_body(D, C0, yf_hbm, gidx_hbm, dst_hbm, se_hbm, part_hbm,
              gidx_v, dst_v, se_v, rows_v, rows2_v, rsc_v, rsc2_v, acc_sh,
              sg0, sg1, ss0, ss1):
    c = lax.axis_index("c")
    s = lax.axis_index("s")
    C1 = (EP // K) // NS - C0
    my_c = jnp.where(c == 0, C0, C1)
    my_base = c * NS * C0 + s * my_c

    for i in range(32):
        for d in range(D // L):
            rsc_v[i, pl.ds(d * L, L)] = jnp.zeros((L,), f32)

    def zero_slice(t, _):
        pltpu.sync_copy(
            rsc_v.at[pl.ds(0, 32)],
            acc_sh.at[pl.ds(pl.multiple_of(s * 640 + t * 32, 8), 32)])
        return 0
    lax.fori_loop(0, 20, zero_slice, 0)

    plsc.subcore_barrier()

    def scale(rv, out, sj):
        for g in range(K // L):
            sv = se_v[sj, pl.ds(g * L, L)]
            for i in range(L):
                sc = sv[i]
                row = g * L + i
                for d in range(D // L):
                    sl = pl.ds(d * L, L)
                    out[row, sl] = rv[row, sl] * sc

    def mblock(mb, _):
        ro = pl.multiple_of(my_base + mb * MB, 8)
        pltpu.sync_copy(gidx_hbm.at[pl.ds(ro, MB)], gidx_v)
        pltpu.sync_copy(dst_hbm.at[pl.ds(ro, MB)], dst_v)
        pltpu.sync_copy(se_hbm.at[pl.ds(ro, MB)], se_v)

        pltpu.async_copy(yf_hbm.at[gidx_v.at[0]], rows_v, sg0)

        def pair(j2, _):
            j0 = j2 * 2
            j1 = j0 + 1
            pltpu.async_copy(yf_hbm.at[gidx_v.at[j1]], rows2_v, sg1)
            pltpu.make_async_copy(yf_hbm.at[gidx_v.at[j0]], rows_v, sg0).wait()

            @pl.when(j2 > 0)
            def _():
                pltpu.make_async_copy(
                    rsc_v, acc_sh.at[dst_v.at[j0 - 2]], ss0).wait()
            scale(rows_v, rsc_v, j0)
            pltpu.async_copy(rsc_v, acc_sh.at[dst_v.at[j0]], ss0, add=True)

            @pl.when(j2 + 1 < MB // 2)
            def _():
                pltpu.async_copy(yf_hbm.at[gidx_v.at[j0 + 2]], rows_v, sg0)
            pltpu.make_async_copy(
                yf_hbm.at[gidx_v.at[j1]], rows2_v, sg1).wait()

            @pl.when(j2 > 0)
            def _():
                pltpu.make_async_copy(
                    rsc2_v, acc_sh.at[dst_v.at[j1 - 2]], ss1).wait()
            scale(rows2_v, rsc2_v, j1)
            pltpu.async_copy(rsc2_v, acc_sh.at[dst_v.at[j1]], ss1, add=True)
            return 0
        lax.fori_loop(0, MB // 2, pair, 0)
        pltpu.make_async_copy(rsc_v, acc_sh.at[dst_v.at[MB - 2]], ss0).wait()
        pltpu.make_async_copy(rsc2_v, acc_sh.at[dst_v.at[MB - 1]], ss1).wait()
        return 0
    lax.fori_loop(0, my_c // MB, mblock, 0)

    plsc.subcore_barrier()
    rbk = pl.multiple_of(s * 640, 8)
    pltpu.sync_copy(acc_sh.at[pl.ds(rbk, 640)],
                    part_hbm.at[c, pl.ds(rbk, 640)])


def _make_agg(D, C0=128):
    @functools.partial(
        pl.kernel,
        out_type=jax.ShapeDtypeStruct((NC, NP, D), f32),
        mesh=_mesh,
        compiler_params=pltpu.CompilerParams(needs_layout_passes=False),
        scratch_types=(
            pltpu.VMEM((MB, K), i32),
            pltpu.VMEM((MB, K), i32),
            pltpu.VMEM((MB, K), f32),
            pltpu.VMEM((K, D128), f32),
            pltpu.VMEM((K, D128), f32),
            pltpu.VMEM((K, D), f32),
            pltpu.VMEM((K, D), f32),
            pltpu.VMEM_SHARED((NP, D), f32),
            pltpu.SemaphoreType.DMA,
            pltpu.SemaphoreType.DMA,
            pltpu.SemaphoreType.DMA,
            pltpu.SemaphoreType.DMA,
        ),
    )
    def agg(yf, gidx, dst, se, part, *rest):
        _agg_body(D, C0, yf, gidx, dst, se, part, *rest)
    return agg


_agg_k = {D: _make_agg(128, C0=192) for D in (128, 64, 32)}


BN = 1000


def _dense_first_body(x_ref, w_ref, r_ref, b_ref, y_ref, xrb_ref):
    xb = x_ref[...]
    xrb_ref[...] = jnp.dot(xb, r_ref[...],
                           preferred_element_type=f32) + b_ref[...]
    for r in range(R_REL):
        y_ref[r] = jnp.dot(xb, w_ref[r], preferred_element_type=f32)


def _dense_next_body(dprev, xrb_ref, p_ref, w_ref, r_ref, b_ref,
                     y_ref, xrb2_ref):
    p = p_ref[0] + p_ref[1]
    if p.shape[1] < D128:
        p = jnp.concatenate(
            [p, jnp.zeros((BN, D128 - p.shape[1]), f32)], axis=1)
    h = jnp.maximum(xrb_ref[...] + p, 0.0)
    xrb2_ref[...] = jnp.dot(h, r_ref[...],
                            preferred_element_type=f32) + b_ref[...]
    for r in range(R_REL):
        y_ref[r] = jnp.dot(h, w_ref[r], preferred_element_type=f32)


def _dense_first(x, W, Rm, b):
    return pl.pallas_call(
        _dense_first_body,
        grid=(N // BN,),
        in_specs=[
            pl.BlockSpec((BN, D128), lambda i: (i, 0)),
            pl.BlockSpec((R_REL, D128, D128), lambda i: (0, 0, 0)),
            pl.BlockSpec((D128, D128), lambda i: (0, 0)),
            pl.BlockSpec((1, D128), lambda i: (0, 0)),
        ],
        out_specs=[
            pl.BlockSpec((R_REL, BN, D128), lambda i: (0, i, 0)),
            pl.BlockSpec((BN, D128), lambda i: (i, 0)),
        ],
        out_shape=[
            jax.ShapeDtypeStruct((R_REL, N, D128), f32),
            jax.ShapeDtypeStruct((N, D128), f32),
        ],
    )(x, W, Rm, b.reshape(1, D128))


def _dense_next(xrb, part, W, Rm, b, dprev):
    return pl.pallas_call(
        functools.partial(_dense_next_body, dprev),
        grid=(N // BN,),
        in_specs=[
            pl.BlockSpec((BN, D128), lambda i: (i, 0)),
            pl.BlockSpec((NC, BN, D128), lambda i: (0, i, 0)),
            pl.BlockSpec((R_REL, D128, D128), lambda i: (0, 0, 0)),
            pl.BlockSpec((D128, D128), lambda i: (0, 0)),
            pl.BlockSpec((1, D128), lambda i: (0, 0)),
        ],
        out_specs=[
            pl.BlockSpec((R_REL, BN, D128), lambda i: (0, i, 0)),
            pl.BlockSpec((BN, D128), lambda i: (i, 0)),
        ],
        out_shape=[
            jax.ShapeDtypeStruct((R_REL, N, D128), f32),
            jax.ShapeDtypeStruct((N, D128), f32),
        ],
    )(xrb, part, W, Rm, b.reshape(1, D128))


def _final_body(xrb_ref, p_ref, fw_ref, fb_ref, o_ref):
    p = p_ref[0] + p_ref[1]
    if p.shape[1] < D128:
        p = jnp.concatenate(
            [p, jnp.zeros((BN, D128 - p.shape[1]), f32)], axis=1)
    h = jnp.maximum(xrb_ref[...] + p, 0.0)
    z = jnp.sum(h * fw_ref[...].reshape(1, D128), axis=1, keepdims=True)
    o_ref[...] = jax.nn.sigmoid(z + fb_ref[...])


def _final(xrb, part, fcW, fcB):
    return pl.pallas_call(
        _final_body,
        grid=(N // BN,),
        in_specs=[
            pl.BlockSpec((BN, D128), lambda i: (i, 0)),
            pl.BlockSpec((NC, BN, D128), lambda i: (0, i, 0)),
            pl.BlockSpec((D128, 1), lambda i: (0, 0)),
            pl.BlockSpec((1, 1), lambda i: (0, 0)),
        ],
        out_specs=pl.BlockSpec((BN, 1), lambda i: (i, 0)),
        out_shape=jax.ShapeDtypeStruct((N, 1), f32),
    )(xrb, part, fcW, fcB.reshape(1, 1))


def _pad2(a, rows, cols):
    return jnp.pad(a, ((0, rows - a.shape[0]), (0, cols - a.shape[1])))


def kernel(x, edge_index, edge_type, W1, R1, B1, W2, R2, B2, W3, R3, B3,
           W4, R4, B4, fcW, fcB):
    src = jnp.pad(edge_index[0].astype(i32), (0, EP - E))
    dst = jnp.pad(edge_index[1].astype(i32), (0, EP - E))
    rel = jnp.pad(edge_type.astype(i32), (0, EP - E))

    gidx, se = _prep(src, dst, rel)
    gidx2 = gidx.reshape(EP // K, K)
    dst2 = dst.reshape(EP // K, K)
    se2 = se.reshape(EP // K, K)

    def padw(W, Rm, b):
        Wp = jnp.stack([_pad2(W[r], D128, D128) for r in range(R_REL)])
        return Wp, _pad2(Rm, D128, D128), jnp.pad(b, (0, D128 - b.shape[0]))

    layers = [padw(W1, R1, B1), padw(W2, R2, B2),
              padw(W3, R3, B3), padw(W4, R4, B4)]

    douts = [128, 64, 32, 32]
    y, xrb = _dense_first(x, *layers[0])
    part = _agg_k[douts[0]](y.reshape(R_REL * N, D128), gidx2, dst2, se2)
    for li, (Wp, Rp, bp) in enumerate(layers[1:], start=1):
        y, xrb = _dense_next(xrb, part, Wp, Rp, bp, douts[li - 1])
        part = _agg_k[douts[li]](y.reshape(R_REL * N, D128), gidx2, dst2, se2)
    fwp = jnp.pad(fcW, ((0, D128 - fcW.shape[0]), (0, 0)))
    return _final(xrb, part, fwp, fcB)


# speedup vs baseline: 1.0813x; 1.0127x over previous
import functools

import jax
import jax.numpy as jnp
from jax import lax
from jax.experimental import pallas as pl
from jax.experimental.pallas import tpu as pltpu
from jax.experimental.pallas import tpu_sc as plsc

N = 10000
E = 320000
R_REL = 3
NC, NS, L = 2, 16, 16
NW = NC * NS
EP = 327680
EPW = EP // NW
EPS = E // NS
CN = R_REL * N
CNP = 30720
CPT = CNP // NS
K = 80
CPW = EPW // K
NP = 10240
D128 = 128
MB = 16
BB = 2000
BB3 = 2048

_mesh = plsc.VectorSubcoreMesh(
    core_axis_name="c", subcore_axis_name="s", num_cores=NC, num_subcores=NS)

f32 = jnp.float32
i32 = jnp.int32


def _prep_body(src_hbm, dst_hbm, rel_hbm,
               gidx_hbm, se_hbm,
               cnt_v, bdst_v, brel_v, bsrc_v, mrg_v, inv_v, oge_v, ose_v,
               cnt_sh, inv_sh, sem):
    c = lax.axis_index("c")
    s = lax.axis_index("s")
    w = c * NS + s

    ones = jnp.ones((L,), f32)

    def zero_cnt(i, _):
        cnt_v[pl.ds(pl.multiple_of(i * L, L), L)] = jnp.zeros((L,), f32)
        return 0
    lax.fori_loop(0, CNP // L, zero_cnt, 0)

    def count_block(b, _):
        base = pl.multiple_of(s * EPS + b * BB, 8)
        pltpu.sync_copy(dst_hbm.at[pl.ds(base, BB)], bdst_v.at[pl.ds(0, BB)])
        pltpu.sync_copy(rel_hbm.at[pl.ds(base, BB)], brel_v.at[pl.ds(0, BB)])

        def count_chunk(i, _):
            o = pl.multiple_of(i * L, L)
            d = bdst_v[pl.ds(o, L)]
            r = brel_v[pl.ds(o, L)]
            cidx = r * N + d
            plsc.addupdate_scatter(cnt_v, [cidx], ones)
            return 0
        lax.fori_loop(0, BB // L, count_chunk, 0)
        return 0
    lax.fori_loop(0, EPS // BB, count_block, 0)

    pltpu.sync_copy(cnt_v, cnt_sh.at[s])
    plsc.subcore_barrier()

    for k2 in range(NS):
        pltpu.sync_copy(cnt_sh.at[k2, pl.ds(pl.multiple_of(s * CPT, 8), CPT)],
                        mrg_v.at[k2])

    def inv_chunk(j, _):
        o = pl.multiple_of(j * L, L)
        tot = mrg_v[0, pl.ds(o, L)]
        for k2 in range(1, NS):
            tot = tot + mrg_v[k2, pl.ds(o, L)]
        inv_v[pl.ds(o, L)] = 1.0 / jnp.maximum(tot, 1.0)
        return 0
    lax.fori_loop(0, CPT // L, inv_chunk, 0)

    pltpu.sync_copy(inv_v,
                    inv_sh.at[pl.ds(pl.multiple_of(s * CPT, 8), CPT)])
    plsc.subcore_barrier()

    inv_full_view = cnt_v
    pltpu.sync_copy(inv_sh, inv_full_view)

    lane = lax.iota(i32, L)

    def out_block(b, _):
        base = pl.multiple_of(w * EPW + b * BB3, 8)
        pltpu.sync_copy(src_hbm.at[pl.ds(base, BB3)], bsrc_v)
        pltpu.sync_copy(dst_hbm.at[pl.ds(base, BB3)], bdst_v)
        pltpu.sync_copy(rel_hbm.at[pl.ds(base, BB3)], brel_v)

        def out_chunk(i, _):
            o = pl.multiple_of(i * L, L)
            sv = bsrc_v[pl.ds(o, L)]
            d = bdst_v[pl.ds(o, L)]
            r = brel_v[pl.ds(o, L)]
            rn = r * N
            se = plsc.load_gather(inv_full_view, [rn + d])
            real = (base + o + lane) < E
            ose_v[pl.ds(o, L)] = jnp.where(real, se, 0.0)
            oge_v[pl.ds(o, L)] = rn + sv
            return 0
        lax.fori_loop(0, BB3 // L, out_chunk, 0)

        pltpu.sync_copy(ose_v, se_hbm.at[pl.ds(base, BB3)])
        pltpu.sync_copy(oge_v, gidx_hbm.at[pl.ds(base, BB3)])
        return 0
    lax.fori_loop(0, EPW // BB3, out_block, 0)


@functools.partial(
    pl.kernel,
    out_type=(jax.ShapeDtypeStruct((EP,), i32),
              jax.ShapeDtypeStruct((EP,), f32)),
    mesh=_mesh,
    compiler_params=pltpu.CompilerParams(needs_layout_passes=False),
    scratch_types=(
        pltpu.VMEM((CNP,), f32),
        pltpu.VMEM((BB3,), i32),
        pltpu.VMEM((BB3,), i32),
        pltpu.VMEM((BB3,), i32),
        pltpu.VMEM((NS, CPT), f32),
        pltpu.VMEM((CPT,), f32),
        pltpu.VMEM((BB3,), i32),
        pltpu.VMEM((BB3,), f32),
        pltpu.VMEM_SHARED((NS, CNP), f32),
        pltpu.VMEM_SHARED((CNP,), f32),
        pltpu.SemaphoreType.DMA,
    ),
)
def _prep(src_hbm, dst_hbm, rel_hbm, gidx_hbm, se_hbm, *rest):
    _prep_body(src_hbm, dst_hbm, rel_hbm, gidx_hbm, se_hbm, *rest)


def _agg_body(D, C0, yf_hbm, gidx_hbm, dst_hbm, se_hbm, part_hbm,
              gidx_v, dst_v, se_v, rows_v, rows2_v, rsc_v, rsc2_v, acc_sh,
              sg0, sg1, ss0, ss1):
    c = lax.axis_index("c")
    s = lax.axis_index("s")
    C1 = (EP // K) // NS - C0
    my_c = jnp.where(c == 0, C0, C1)
    my_base = c * NS * C0 + s * my_c

    for i in range(32):
        for d in range(D // L):
            rsc_v[i, pl.ds(d * L, L)] = jnp.zeros((L,), f32)

    def zero_slice(t, _):
        pltpu.sync_copy(
            rsc_v.at[pl.ds(0, 32)],
            acc_sh.at[pl.ds(pl.multiple_of(s * 640 + t * 32, 8), 32)])
        return 0
    lax.fori_loop(0, 20, zero_slice, 0)

    plsc.subcore_barrier()

    def scale(rv, out, sj):
        for g in range(K // L):
            sv = se_v[sj, pl.ds(g * L, L)]
            for i in range(L):
                sc = sv[i]
                row = g * L + i
                for d in range(D // L):
                    sl = pl.ds(d * L, L)
                    out[row, sl] = rv[row, sl] * sc

    def mblock(mb, _):
        ro = pl.multiple_of(my_base + mb * MB, 8)
        pltpu.sync_copy(gidx_hbm.at[pl.ds(ro, MB)], gidx_v)
        pltpu.sync_copy(dst_hbm.at[pl.ds(ro, MB)], dst_v)
        pltpu.sync_copy(se_hbm.at[pl.ds(ro, MB)], se_v)

        pltpu.async_copy(yf_hbm.at[gidx_v.at[0]], rows_v, sg0)

        def pair(j2, _):
            j0 = j2 * 2
            j1 = j0 + 1
            pltpu.async_copy(yf_hbm.at[gidx_v.at[j1]], rows2_v, sg1)
            pltpu.make_async_copy(yf_hbm.at[gidx_v.at[j0]], rows_v, sg0).wait()

            @pl.when(j2 > 0)
            def _():
                pltpu.make_async_copy(
                    rsc_v, acc_sh.at[dst_v.at[j0 - 2]], ss0).wait()
            scale(rows_v, rsc_v, j0)
            pltpu.async_copy(rsc_v, acc_sh.at[dst_v.at[j0]], ss0, add=True)

            @pl.when(j2 + 1 < MB // 2)
            def _():
                pltpu.async_copy(yf_hbm.at[gidx_v.at[j0 + 2]], rows_v, sg0)
            pltpu.make_async_copy(
                yf_hbm.at[gidx_v.at[j1]], rows2_v, sg1).wait()

            @pl.when(j2 > 0)
            def _():
                pltpu.make_async_copy(
                    rsc2_v, acc_sh.at[dst_v.at[j1 - 2]], ss1).wait()
            scale(rows2_v, rsc2_v, j1)
            pltpu.async_copy(rsc2_v, acc_sh.at[dst_v.at[j1]], ss1, add=True)
            return 0
        lax.fori_loop(0, MB // 2, pair, 0)
        pltpu.make_async_copy(rsc_v, acc_sh.at[dst_v.at[MB - 2]], ss0).wait()
        pltpu.make_async_copy(rsc2_v, acc_sh.at[dst_v.at[MB - 1]], ss1).wait()
        return 0
    lax.fori_loop(0, my_c // MB, mblock, 0)

    plsc.subcore_barrier()
    rbk = pl.multiple_of(s * 640, 8)
    pltpu.sync_copy(acc_sh.at[pl.ds(rbk, 640)],
                    part_hbm.at[c, pl.ds(rbk, 640)])


def _make_agg(D, C0=128):
    @functools.partial(
        pl.kernel,
        out_type=jax.ShapeDtypeStruct((NC, NP, D), f32),
        mesh=_mesh,
        compiler_params=pltpu.CompilerParams(needs_layout_passes=False),
        scratch_types=(
            pltpu.VMEM((MB, K), i32),
            pltpu.VMEM((MB, K), i32),
            pltpu.VMEM((MB, K), f32),
            pltpu.VMEM((K, D128), f32),
            pltpu.VMEM((K, D128), f32),
            pltpu.VMEM((K, D), f32),
            pltpu.VMEM((K, D), f32),
            pltpu.VMEM_SHARED((NP, D), f32),
            pltpu.SemaphoreType.DMA,
            pltpu.SemaphoreType.DMA,
            pltpu.SemaphoreType.DMA,
            pltpu.SemaphoreType.DMA,
        ),
    )
    def agg(yf, gidx, dst, se, part, *rest):
        _agg_body(D, C0, yf, gidx, dst, se, part, *rest)
    return agg


_agg_k = {D: _make_agg(128, C0=208) for D in (128, 64, 32)}


BN = 1000


def _dense_first_body(x_ref, w_ref, r_ref, b_ref, y_ref, xrb_ref):
    xb = x_ref[...]
    xrb_ref[...] = jnp.dot(xb, r_ref[...],
                           preferred_element_type=f32) + b_ref[...]
    for r in range(R_REL):
        y_ref[r] = jnp.dot(xb, w_ref[r], preferred_element_type=f32)


def _dense_next_body(dprev, xrb_ref, p_ref, w_ref, r_ref, b_ref,
                     y_ref, xrb2_ref):
    p = p_ref[0] + p_ref[1]
    if p.shape[1] < D128:
        p = jnp.concatenate(
            [p, jnp.zeros((BN, D128 - p.shape[1]), f32)], axis=1)
    h = jnp.maximum(xrb_ref[...] + p, 0.0)
    xrb2_ref[...] = jnp.dot(h, r_ref[...],
                            preferred_element_type=f32) + b_ref[...]
    for r in range(R_REL):
        y_ref[r] = jnp.dot(h, w_ref[r], preferred_element_type=f32)


def _dense_first(x, W, Rm, b):
    return pl.pallas_call(
        _dense_first_body,
        grid=(N // BN,),
        in_specs=[
            pl.BlockSpec((BN, D128), lambda i: (i, 0)),
            pl.BlockSpec((R_REL, D128, D128), lambda i: (0, 0, 0)),
            pl.BlockSpec((D128, D128), lambda i: (0, 0)),
            pl.BlockSpec((1, D128), lambda i: (0, 0)),
        ],
        out_specs=[
            pl.BlockSpec((R_REL, BN, D128), lambda i: (0, i, 0)),
            pl.BlockSpec((BN, D128), lambda i: (i, 0)),
        ],
        out_shape=[
            jax.ShapeDtypeStruct((R_REL, N, D128), f32),
            jax.ShapeDtypeStruct((N, D128), f32),
        ],
    )(x, W, Rm, b.reshape(1, D128))


def _dense_next(xrb, part, W, Rm, b, dprev):
    return pl.pallas_call(
        functools.partial(_dense_next_body, dprev),
        grid=(N // BN,),
        in_specs=[
            pl.BlockSpec((BN, D128), lambda i: (i, 0)),
            pl.BlockSpec((NC, BN, D128), lambda i: (0, i, 0)),
            pl.BlockSpec((R_REL, D128, D128), lambda i: (0, 0, 0)),
            pl.BlockSpec((D128, D128), lambda i: (0, 0)),
            pl.BlockSpec((1, D128), lambda i: (0, 0)),
        ],
        out_specs=[
            pl.BlockSpec((R_REL, BN, D128), lambda i: (0, i, 0)),
            pl.BlockSpec((BN, D128), lambda i: (i, 0)),
        ],
        out_shape=[
            jax.ShapeDtypeStruct((R_REL, N, D128), f32),
            jax.ShapeDtypeStruct((N, D128), f32),
        ],
    )(xrb, part, W, Rm, b.reshape(1, D128))


def _final_body(xrb_ref, p_ref, fw_ref, fb_ref, o_ref):
    p = p_ref[0] + p_ref[1]
    if p.shape[1] < D128:
        p = jnp.concatenate(
            [p, jnp.zeros((BN, D128 - p.shape[1]), f32)], axis=1)
    h = jnp.maximum(xrb_ref[...] + p, 0.0)
    z = jnp.sum(h * fw_ref[...].reshape(1, D128), axis=1, keepdims=True)
    o_ref[...] = jax.nn.sigmoid(z + fb_ref[...])


def _final(xrb, part, fcW, fcB):
    return pl.pallas_call(
        _final_body,
        grid=(N // BN,),
        in_specs=[
            pl.BlockSpec((BN, D128), lambda i: (i, 0)),
            pl.BlockSpec((NC, BN, D128), lambda i: (0, i, 0)),
            pl.BlockSpec((D128, 1), lambda i: (0, 0)),
            pl.BlockSpec((1, 1), lambda i: (0, 0)),
        ],
        out_specs=pl.BlockSpec((BN, 1), lambda i: (i, 0)),
        out_shape=jax.ShapeDtypeStruct((N, 1), f32),
    )(xrb, part, fcW, fcB.reshape(1, 1))


def _pad2(a, rows, cols):
    return jnp.pad(a, ((0, rows - a.shape[0]), (0, cols - a.shape[1])))


def kernel(x, edge_index, edge_type, W1, R1, B1, W2, R2, B2, W3, R3, B3,
           W4, R4, B4, fcW, fcB):
    src = jnp.pad(edge_index[0].astype(i32), (0, EP - E))
    dst = jnp.pad(edge_index[1].astype(i32), (0, EP - E))
    rel = jnp.pad(edge_type.astype(i32), (0, EP - E))

    gidx, se = _prep(src, dst, rel)
    gidx2 = gidx.reshape(EP // K, K)
    dst2 = dst.reshape(EP // K, K)
    se2 = se.reshape(EP // K, K)

    def padw(W, Rm, b):
        Wp = jnp.stack([_pad2(W[r], D128, D128) for r in range(R_REL)])
        return Wp, _pad2(Rm, D128, D128), jnp.pad(b, (0, D128 - b.shape[0]))

    layers = [padw(W1, R1, B1), padw(W2, R2, B2),
              padw(W3, R3, B3), padw(W4, R4, B4)]

    douts = [128, 64, 32, 32]
    y, xrb = _dense_first(x, *layers[0])
    part = _agg_k[douts[0]](y.reshape(R_REL * N, D128), gidx2, dst2, se2)
    for li, (Wp, Rp, bp) in enumerate(layers[1:], start=1):
        y, xrb = _dense_next(xrb, part, Wp, Rp, bp, douts[li - 1])
        part = _agg_k[douts[li]](y.reshape(R_REL * N, D128), gidx2, dst2, se2)
    fwp = jnp.pad(fcW, ((0, D128 - fcW.shape[0]), (0, 0)))
    return _final(xrb, part, fwp, fcB)


# speedup vs baseline: 1.0944x; 1.0122x over previous
import functools

import jax
import jax.numpy as jnp
from jax import lax
from jax.experimental import pallas as pl
from jax.experimental.pallas import tpu as pltpu
from jax.experimental.pallas import tpu_sc as plsc

N = 10000
E = 320000
R_REL = 3
NC, NS, L = 2, 16, 16
NW = NC * NS
EP = 327680
EPW = EP // NW
EPS = E // NS
CN = R_REL * N
CNP = 30720
CPT = CNP // NS
K = 80
CPW = EPW // K
NP = 10240
D128 = 128
MB = 16
BB = 2000
BB3 = 2048

_mesh = plsc.VectorSubcoreMesh(
    core_axis_name="c", subcore_axis_name="s", num_cores=NC, num_subcores=NS)

f32 = jnp.float32
i32 = jnp.int32


def _prep_body(src_hbm, dst_hbm, rel_hbm,
               gidx_hbm, se_hbm,
               cnt_v, bdst_v, brel_v, bsrc_v, mrg_v, inv_v, oge_v, ose_v,
               cnt_sh, inv_sh, sem):
    c = lax.axis_index("c")
    s = lax.axis_index("s")
    w = c * NS + s

    ones = jnp.ones((L,), f32)

    def zero_cnt(i, _):
        cnt_v[pl.ds(pl.multiple_of(i * L, L), L)] = jnp.zeros((L,), f32)
        return 0
    lax.fori_loop(0, CNP // L, zero_cnt, 0)

    def count_block(b, _):
        base = pl.multiple_of(s * EPS + b * BB, 8)
        pltpu.sync_copy(dst_hbm.at[pl.ds(base, BB)], bdst_v.at[pl.ds(0, BB)])
        pltpu.sync_copy(rel_hbm.at[pl.ds(base, BB)], brel_v.at[pl.ds(0, BB)])

        def count_chunk(i, _):
            o = pl.multiple_of(i * L, L)
            d = bdst_v[pl.ds(o, L)]
            r = brel_v[pl.ds(o, L)]
            cidx = r * N + d
            plsc.addupdate_scatter(cnt_v, [cidx], ones)
            return 0
        lax.fori_loop(0, BB // L, count_chunk, 0)
        return 0
    lax.fori_loop(0, EPS // BB, count_block, 0)

    pltpu.sync_copy(cnt_v, cnt_sh.at[s])
    plsc.subcore_barrier()

    for k2 in range(NS):
        pltpu.sync_copy(cnt_sh.at[k2, pl.ds(pl.multiple_of(s * CPT, 8), CPT)],
                        mrg_v.at[k2])

    def inv_chunk(j, _):
        o = pl.multiple_of(j * L, L)
        tot = mrg_v[0, pl.ds(o, L)]
        for k2 in range(1, NS):
            tot = tot + mrg_v[k2, pl.ds(o, L)]
        inv_v[pl.ds(o, L)] = 1.0 / jnp.maximum(tot, 1.0)
        return 0
    lax.fori_loop(0, CPT // L, inv_chunk, 0)

    pltpu.sync_copy(inv_v,
                    inv_sh.at[pl.ds(pl.multiple_of(s * CPT, 8), CPT)])
    plsc.subcore_barrier()

    inv_full_view = cnt_v
    pltpu.sync_copy(inv_sh, inv_full_view)

    lane = lax.iota(i32, L)

    def out_block(b, _):
        base = pl.multiple_of(w * EPW + b * BB3, 8)
        pltpu.sync_copy(src_hbm.at[pl.ds(base, BB3)], bsrc_v)
        pltpu.sync_copy(dst_hbm.at[pl.ds(base, BB3)], bdst_v)
        pltpu.sync_copy(rel_hbm.at[pl.ds(base, BB3)], brel_v)

        def out_chunk(i, _):
            o = pl.multiple_of(i * L, L)
            sv = bsrc_v[pl.ds(o, L)]
            d = bdst_v[pl.ds(o, L)]
            r = brel_v[pl.ds(o, L)]
            rn = r * N
            se = plsc.load_gather(inv_full_view, [rn + d])
            real = (base + o + lane) < E
            ose_v[pl.ds(o, L)] = jnp.where(real, se, 0.0)
            oge_v[pl.ds(o, L)] = rn + sv
            return 0
        lax.fori_loop(0, BB3 // L, out_chunk, 0)

        pltpu.sync_copy(ose_v, se_hbm.at[pl.ds(base, BB3)])
        pltpu.sync_copy(oge_v, gidx_hbm.at[pl.ds(base, BB3)])
        return 0
    lax.fori_loop(0, EPW // BB3, out_block, 0)


@functools.partial(
    pl.kernel,
    out_type=(jax.ShapeDtypeStruct((EP,), i32),
              jax.ShapeDtypeStruct((EP,), f32)),
    mesh=_mesh,
    compiler_params=pltpu.CompilerParams(needs_layout_passes=False),
    scratch_types=(
        pltpu.VMEM((CNP,), f32),
        pltpu.VMEM((BB3,), i32),
        pltpu.VMEM((BB3,), i32),
        pltpu.VMEM((BB3,), i32),
        pltpu.VMEM((NS, CPT), f32),
        pltpu.VMEM((CPT,), f32),
        pltpu.VMEM((BB3,), i32),
        pltpu.VMEM((BB3,), f32),
        pltpu.VMEM_SHARED((NS, CNP), f32),
        pltpu.VMEM_SHARED((CNP,), f32),
        pltpu.SemaphoreType.DMA,
    ),
)
def _prep(src_hbm, dst_hbm, rel_hbm, gidx_hbm, se_hbm, *rest):
    _prep_body(src_hbm, dst_hbm, rel_hbm, gidx_hbm, se_hbm, *rest)


def _agg_body(D, C0, yf_hbm, gidx_hbm, dst_hbm, se_hbm, part_hbm,
              gidx_v, dst_v, se_v, rows_v, rows2_v, rsc_v, rsc2_v, acc_sh,
              sg0, sg1, ss0, ss1):
    c = lax.axis_index("c")
    s = lax.axis_index("s")
    C1 = (EP // K) // NS - C0
    my_c = jnp.where(c == 0, C0, C1)
    my_base = c * NS * C0 + s * my_c

    for i in range(32):
        for d in range(D // L):
            rsc_v[i, pl.ds(d * L, L)] = jnp.zeros((L,), f32)

    def zero_slice(t, _):
        pltpu.sync_copy(
            rsc_v.at[pl.ds(0, 32)],
            acc_sh.at[pl.ds(pl.multiple_of(s * 640 + t * 32, 8), 32)])
        return 0
    lax.fori_loop(0, 20, zero_slice, 0)

    plsc.subcore_barrier()

    def scale(rv, out, sj):
        for g in range(K // L):
            sv = se_v[sj, pl.ds(g * L, L)]
            for i in range(L):
                sc = sv[i]
                row = g * L + i
                for d in range(D // L):
                    sl = pl.ds(d * L, L)
                    out[row, sl] = rv[row, sl] * sc

    def mblock(mb, _):
        ro = pl.multiple_of(my_base + mb * MB, 8)
        pltpu.sync_copy(gidx_hbm.at[pl.ds(ro, MB)], gidx_v)
        pltpu.sync_copy(dst_hbm.at[pl.ds(ro, MB)], dst_v)
        pltpu.sync_copy(se_hbm.at[pl.ds(ro, MB)], se_v)

        pltpu.async_copy(yf_hbm.at[gidx_v.at[0]], rows_v, sg0)

        def pair(j2, _):
            j0 = j2 * 2
            j1 = j0 + 1
            pltpu.async_copy(yf_hbm.at[gidx_v.at[j1]], rows2_v, sg1)
            pltpu.make_async_copy(yf_hbm.at[gidx_v.at[j0]], rows_v, sg0).wait()

            @pl.when(j2 > 0)
            def _():
                pltpu.make_async_copy(
                    rsc_v, acc_sh.at[dst_v.at[j0 - 2]], ss0).wait()
            scale(rows_v, rsc_v, j0)
            pltpu.async_copy(rsc_v, acc_sh.at[dst_v.at[j0]], ss0, add=True)

            @pl.when(j2 + 1 < MB // 2)
            def _():
                pltpu.async_copy(yf_hbm.at[gidx_v.at[j0 + 2]], rows_v, sg0)
            pltpu.make_async_copy(
                yf_hbm.at[gidx_v.at[j1]], rows2_v, sg1).wait()

            @pl.when(j2 > 0)
            def _():
                pltpu.make_async_copy(
                    rsc2_v, acc_sh.at[dst_v.at[j1 - 2]], ss1).wait()
            scale(rows2_v, rsc2_v, j1)
            pltpu.async_copy(rsc2_v, acc_sh.at[dst_v.at[j1]], ss1, add=True)
            return 0
        lax.fori_loop(0, MB // 2, pair, 0)
        pltpu.make_async_copy(rsc_v, acc_sh.at[dst_v.at[MB - 2]], ss0).wait()
        pltpu.make_async_copy(rsc2_v, acc_sh.at[dst_v.at[MB - 1]], ss1).wait()
        return 0
    lax.fori_loop(0, my_c // MB, mblock, 0)

    plsc.subcore_barrier()
    rbk = pl.multiple_of(s * 640, 8)
    pltpu.sync_copy(acc_sh.at[pl.ds(rbk, 640)],
                    part_hbm.at[c, pl.ds(rbk, 640)])


def _make_agg(D, C0=128):
    @functools.partial(
        pl.kernel,
        out_type=jax.ShapeDtypeStruct((NC, NP, D), f32),
        mesh=_mesh,
        compiler_params=pltpu.CompilerParams(needs_layout_passes=False),
        scratch_types=(
            pltpu.VMEM((MB, K), i32),
            pltpu.VMEM((MB, K), i32),
            pltpu.VMEM((MB, K), f32),
            pltpu.VMEM((K, D128), f32),
            pltpu.VMEM((K, D128), f32),
            pltpu.VMEM((K, D), f32),
            pltpu.VMEM((K, D), f32),
            pltpu.VMEM_SHARED((NP, D), f32),
            pltpu.SemaphoreType.DMA,
            pltpu.SemaphoreType.DMA,
            pltpu.SemaphoreType.DMA,
            pltpu.SemaphoreType.DMA,
        ),
    )
    def agg(yf, gidx, dst, se, part, *rest):
        _agg_body(D, C0, yf, gidx, dst, se, part, *rest)
    return agg


_agg_k = {D: _make_agg(128, C0=224) for D in (128, 64, 32)}


BN = 1000


def _dense_first_body(x_ref, w_ref, r_ref, b_ref, y_ref, xrb_ref):
    xb = x_ref[...]
    xrb_ref[...] = jnp.dot(xb, r_ref[...],
                           preferred_element_type=f32) + b_ref[...]
    for r in range(R_REL):
        y_ref[r] = jnp.dot(xb, w_ref[r], preferred_element_type=f32)


def _dense_next_body(dprev, xrb_ref, p_ref, w_ref, r_ref, b_ref,
                     y_ref, xrb2_ref):
    p = p_ref[0] + p_ref[1]
    if p.shape[1] < D128:
        p = jnp.concatenate(
            [p, jnp.zeros((BN, D128 - p.shape[1]), f32)], axis=1)
    h = jnp.maximum(xrb_ref[...] + p, 0.0)
    xrb2_ref[...] = jnp.dot(h, r_ref[...],
                            preferred_element_type=f32) + b_ref[...]
    for r in range(R_REL):
        y_ref[r] = jnp.dot(h, w_ref[r], preferred_element_type=f32)


def _dense_first(x, W, Rm, b):
    return pl.pallas_call(
        _dense_first_body,
        grid=(N // BN,),
        in_specs=[
            pl.BlockSpec((BN, D128), lambda i: (i, 0)),
            pl.BlockSpec((R_REL, D128, D128), lambda i: (0, 0, 0)),
            pl.BlockSpec((D128, D128), lambda i: (0, 0)),
            pl.BlockSpec((1, D128), lambda i: (0, 0)),
        ],
        out_specs=[
            pl.BlockSpec((R_REL, BN, D128), lambda i: (0, i, 0)),
            pl.BlockSpec((BN, D128), lambda i: (i, 0)),
        ],
        out_shape=[
            jax.ShapeDtypeStruct((R_REL, N, D128), f32),
            jax.ShapeDtypeStruct((N, D128), f32),
        ],
    )(x, W, Rm, b.reshape(1, D128))


def _dense_next(xrb, part, W, Rm, b, dprev):
    return pl.pallas_call(
        functools.partial(_dense_next_body, dprev),
        grid=(N // BN,),
        in_specs=[
            pl.BlockSpec((BN, D128), lambda i: (i, 0)),
            pl.BlockSpec((NC, BN, D128), lambda i: (0, i, 0)),
            pl.BlockSpec((R_REL, D128, D128), lambda i: (0, 0, 0)),
            pl.BlockSpec((D128, D128), lambda i: (0, 0)),
            pl.BlockSpec((1, D128), lambda i: (0, 0)),
        ],
        out_specs=[
            pl.BlockSpec((R_REL, BN, D128), lambda i: (0, i, 0)),
            pl.BlockSpec((BN, D128), lambda i: (i, 0)),
        ],
        out_shape=[
            jax.ShapeDtypeStruct((R_REL, N, D128), f32),
            jax.ShapeDtypeStruct((N, D128), f32),
        ],
    )(xrb, part, W, Rm, b.reshape(1, D128))


def _final_body(xrb_ref, p_ref, fw_ref, fb_ref, o_ref):
    p = p_ref[0] + p_ref[1]
    if p.shape[1] < D128:
        p = jnp.concatenate(
            [p, jnp.zeros((BN, D128 - p.shape[1]), f32)], axis=1)
    h = jnp.maximum(xrb_ref[...] + p, 0.0)
    z = jnp.sum(h * fw_ref[...].reshape(1, D128), axis=1, keepdims=True)
    o_ref[...] = jax.nn.sigmoid(z + fb_ref[...])


def _final(xrb, part, fcW, fcB):
    return pl.pallas_call(
        _final_body,
        grid=(N // BN,),
        in_specs=[
            pl.BlockSpec((BN, D128), lambda i: (i, 0)),
            pl.BlockSpec((NC, BN, D128), lambda i: (0, i, 0)),
            pl.BlockSpec((D128, 1), lambda i: (0, 0)),
            pl.BlockSpec((1, 1), lambda i: (0, 0)),
        ],
        out_specs=pl.BlockSpec((BN, 1), lambda i: (i, 0)),
        out_shape=jax.ShapeDtypeStruct((N, 1), f32),
    )(xrb, part, fcW, fcB.reshape(1, 1))


def _pad2(a, rows, cols):
    return jnp.pad(a, ((0, rows - a.shape[0]), (0, cols - a.shape[1])))


def kernel(x, edge_index, edge_type, W1, R1, B1, W2, R2, B2, W3, R3, B3,
           W4, R4, B4, fcW, fcB):
    src = jnp.pad(edge_index[0].astype(i32), (0, EP - E))
    dst = jnp.pad(edge_index[1].astype(i32), (0, EP - E))
    rel = jnp.pad(edge_type.astype(i32), (0, EP - E))

    gidx, se = _prep(src, dst, rel)
    gidx2 = gidx.reshape(EP // K, K)
    dst2 = dst.reshape(EP // K, K)
    se2 = se.reshape(EP // K, K)

    def padw(W, Rm, b):
        Wp = jnp.stack([_pad2(W[r], D128, D128) for r in range(R_REL)])
        return Wp, _pad2(Rm, D128, D128), jnp.pad(b, (0, D128 - b.shape[0]))

    layers = [padw(W1, R1, B1), padw(W2, R2, B2),
              padw(W3, R3, B3), padw(W4, R4, B4)]

    douts = [128, 64, 32, 32]
    y, xrb = _dense_first(x, *layers[0])
    part = _agg_k[douts[0]](y.reshape(R_REL * N, D128), gidx2, dst2, se2)
    for li, (Wp, Rp, bp) in enumerate(layers[1:], start=1):
        y, xrb = _dense_next(xrb, part, Wp, Rp, bp, douts[li - 1])
        part = _agg_k[douts[li]](y.reshape(R_REL * N, D128), gidx2, dst2, se2)
    fwp = jnp.pad(fcW, ((0, D128 - fcW.shape[0]), (0, 0)))
    return _final(xrb, part, fwp, fcB)
